# baseline (speedup 1.0000x reference)
import jax
import jax.numpy as jnp
from jax.experimental import pallas as pl

NCLUSTER = 512
PTS = 512


def _body(x_ref, w_ref, out_ref):
    s = jax.lax.dot_general(
        x_ref[...], w_ref[...],
        (((1,), (0,)), ((), ())),
        preferred_element_type=jnp.float32,
        precision=jax.lax.Precision.HIGHEST,
    )
    a = jnp.argmin(s, axis=-1).astype(jnp.int32)
    out_ref[0, 0, :] = a


def kernel(x, C):
    bs, c, h, w = x.shape
    n = bs * h * w
    xt = x.reshape(bs, c, h * w).transpose(0, 2, 1).reshape(n, c)
    xa = jnp.concatenate([xt, jnp.ones((n, 1), jnp.float32)], axis=1)
    wt = jnp.concatenate([-2.0 * C, (C * C).sum(1, keepdims=True)], axis=1).T
    grid = n // PTS
    out = pl.pallas_call(
        _body,
        grid=(grid,),
        in_specs=[
            pl.BlockSpec((PTS, c + 1), lambda i: (i, 0)),
            pl.BlockSpec((c + 1, NCLUSTER), lambda i: (0, 0)),
        ],
        out_specs=pl.BlockSpec((1, 1, PTS), lambda i: (i, 0, 0)),
        out_shape=jax.ShapeDtypeStruct((grid, 1, PTS), jnp.int32),
    )(xa, wt)
    return out.reshape(bs, h * w)


# speedup vs baseline: 2.5232x; 2.5232x over previous
import jax
import jax.numpy as jnp
from jax.experimental import pallas as pl

NCLUSTER = 512
PTS = 512


def _body(x_ref, c_ref, out_ref):
    x0 = x_ref[0, 0:1, :]
    x1 = x_ref[0, 1:2, :]
    x2 = x_ref[0, 2:3, :]
    c0 = c_ref[:, 0:1]
    c1 = c_ref[:, 1:2]
    c2 = c_ref[:, 2:3]
    d = (x0 - c0) ** 2 + (x1 - c1) ** 2 + (x2 - c2) ** 2
    a = jnp.argmin(d, axis=0).astype(jnp.int32)
    out_ref[0, 0, 0, :] = a


def kernel(x, C):
    bs, c, h, w = x.shape
    hw = h * w
    xr = x.reshape(bs, c, hw)
    nj = hw // PTS
    out = pl.pallas_call(
        _body,
        grid=(bs, nj),
        in_specs=[
            pl.BlockSpec((1, c, PTS), lambda i, j: (i, 0, j)),
            pl.BlockSpec((NCLUSTER, c), lambda i, j: (0, 0)),
        ],
        out_specs=pl.BlockSpec((1, 1, 1, PTS), lambda i, j: (i, j, 0, 0)),
        out_shape=jax.ShapeDtypeStruct((bs, nj, 1, PTS), jnp.int32),
    )(xr, C)
    return out.reshape(bs, hw)


# speedup vs baseline: 2.9548x; 1.1710x over previous
import jax
import jax.numpy as jnp
from jax.experimental import pallas as pl

NCLUSTER = 512
PTS = 2048


def _body(x_ref, c_ref, out_ref):
    x0 = x_ref[0, 0:1, :]
    x1 = x_ref[0, 1:2, :]
    x2 = x_ref[0, 2:3, :]
    c0 = c_ref[:, 0:1]
    c1 = c_ref[:, 1:2]
    c2 = c_ref[:, 2:3]
    d = (x0 - c0) ** 2 + (x1 - c1) ** 2 + (x2 - c2) ** 2
    a = jnp.argmin(d, axis=0).astype(jnp.int32)
    out_ref[0, 0, 0, :] = a


def kernel(x, C):
    bs, c, h, w = x.shape
    hw = h * w
    xr = x.reshape(bs, c, hw)
    nj = hw // PTS
    out = pl.pallas_call(
        _body,
        grid=(bs, nj),
        in_specs=[
            pl.BlockSpec((1, c, PTS), lambda i, j: (i, 0, j)),
            pl.BlockSpec((NCLUSTER, c), lambda i, j: (0, 0)),
        ],
        out_specs=pl.BlockSpec((1, 1, 1, PTS), lambda i, j: (i, j, 0, 0)),
        out_shape=jax.ShapeDtypeStruct((bs, nj, 1, PTS), jnp.int32),
    )(xr, C)
    return out.reshape(bs, hw)


# speedup vs baseline: 3.2492x; 1.0997x over previous
import jax
import jax.numpy as jnp
from jax import lax
from jax.experimental import pallas as pl

NCLUSTER = 512
PTS = 2048


def _body(x_ref, w_ref, out_ref):
    x0 = x_ref[0, 0:1, :]
    x1 = x_ref[0, 1:2, :]
    x2 = x_ref[0, 2:3, :]
    w0 = w_ref[:, 0:1]
    w1 = w_ref[:, 1:2]
    w2 = w_ref[:, 2:3]
    cn = w_ref[:, 3:4]
    s = ((cn + w0 * x0) + w1 * x1) + w2 * x2
    a = jnp.argmin(s, axis=0).astype(jnp.int32)
    out_ref[0, 0, 0, :] = a


def kernel(x, C):
    bs, c, h, w = x.shape
    hw = h * w
    xr = x.reshape(bs, c, hw)
    wc = jnp.concatenate([-2.0 * C, (C * C).sum(1, keepdims=True)], axis=1)
    nj = hw // PTS
    out = pl.pallas_call(
        _body,
        grid=(bs, nj),
        in_specs=[
            pl.BlockSpec((1, c, PTS), lambda i, j: (i, 0, j)),
            pl.BlockSpec((NCLUSTER, c + 1), lambda i, j: (0, 0)),
        ],
        out_specs=pl.BlockSpec((1, 1, 1, PTS), lambda i, j: (i, j, 0, 0)),
        out_shape=jax.ShapeDtypeStruct((bs, nj, 1, PTS), jnp.int32),
    )(xr, wc)
    return out.reshape(bs, hw)
